# baseline (speedup 1.0000x reference)
import functools

import jax
import jax.numpy as jnp
from jax import lax
from jax.experimental import pallas as pl
from jax.experimental.pallas import tpu as pltpu
from jax.experimental.pallas import tpu_sc as plsc

_N = 100000
_H = 64
_TXN_IN = 256

_SLOT = 1024
_HALF = 512
_SUB = 128
_NPAD = 100352
_NSLOTS = _NPAD // _SLOT
_NW = 32

_sc_mesh = plsc.VectorSubcoreMesh(core_axis_name="c", subcore_axis_name="s")


@functools.partial(
    pl.kernel,
    mesh=_sc_mesh,
    out_type=[jax.ShapeDtypeStruct((_NPAD, 128), jnp.float32)] * 3,
    scratch_types=[
        pltpu.VMEM((8, _SUB), jnp.int32),
        pltpu.VMEM((_HALF, 128), jnp.float32),
        pltpu.SemaphoreType.DMA,
    ],
)
def _gather3(idx_c, idx_a, idx_e, mem_c, mem_a, mem_e,
             out_c, out_a, out_e, idx_v, rows_v, sem):
    wid = lax.axis_index("s") * 2 + lax.axis_index("c")
    for idx_hbm, mem_hbm, out_hbm in ((idx_c, mem_c, out_c),
                                      (idx_a, mem_a, out_a),
                                      (idx_e, mem_e, out_e)):
        for j in range((_NSLOTS + _NW - 1) // _NW):
            c = wid + _NW * j

            @pl.when(c < _NSLOTS)
            def _():
                pltpu.sync_copy(idx_hbm.at[pl.ds(c * 8, 8)], idx_v)
                for h in range(2):
                    cps = [
                        pltpu.async_copy(
                            mem_hbm.at[idx_v.at[4 * h + q]],
                            rows_v.at[pl.ds(q * _SUB, _SUB)],
                            sem,
                        )
                        for q in range(4)
                    ]
                    for cp in cps:
                        cp.wait()
                    pltpu.sync_copy(
                        rows_v,
                        out_hbm.at[pl.ds(c * _SLOT + h * _HALF, _HALF)])


_BR = 1000


def _pick_row(g2, p2):
    return jnp.where(p2 == 0, g2[:, :_H], g2[:, _H:]).astype(jnp.bfloat16)


def _mlp_body(x_ref, gc_ref, ga_ref, ge_ref, pc_ref, pa_ref, pe_ref,
              w1_ref, b1_ref, wv1_ref, bv1_ref, wv2_ref, bv2_ref, out_ref):
    x = x_ref[...].astype(jnp.bfloat16)
    h = jnp.maximum(
        jnp.dot(x, w1_ref[...], preferred_element_type=jnp.float32) + b1_ref[...],
        0.0).astype(jnp.bfloat16)
    acc = jnp.dot(h, wv1_ref[0:_H, :], preferred_element_type=jnp.float32)
    for k, (g_ref, p_ref) in enumerate(
            ((gc_ref, pc_ref), (ga_ref, pa_ref), (ge_ref, pe_ref))):
        g = _pick_row(g_ref[...], p_ref[0])
        acc += jnp.dot(g, wv1_ref[(k + 1) * _H:(k + 2) * _H, :],
                       preferred_element_type=jnp.float32)
    z = jnp.maximum(acc + bv1_ref[...], 0.0)
    out_ref[...] = (
        jnp.dot(z, wv2_ref[...], preferred_element_type=jnp.float32)
        + bv2_ref[...])


def _mlp(txn_x, gc, ga, ge, pc, pa, pe, W1, b1, Wv1p, bv1, Wv2, bv2):
    grid = _N // _BR
    g_spec = pl.BlockSpec((_BR, 128), lambda i: (i, 0))
    p_spec = pl.BlockSpec((1, _BR, 1), lambda i: (i, 0, 0))
    return pl.pallas_call(
        _mlp_body,
        grid=(grid,),
        in_specs=[
            pl.BlockSpec((_BR, _TXN_IN), lambda i: (i, 0)),
            g_spec, g_spec, g_spec,
            p_spec, p_spec, p_spec,
            pl.BlockSpec((_TXN_IN, _H), lambda i: (0, 0)),
            pl.BlockSpec((1, _H), lambda i: (0, 0)),
            pl.BlockSpec((4 * _H, _H), lambda i: (0, 0)),
            pl.BlockSpec((1, _H), lambda i: (0, 0)),
            pl.BlockSpec((_H, 1), lambda i: (0, 0)),
            pl.BlockSpec((1, 1), lambda i: (0, 0)),
        ],
        out_specs=pl.BlockSpec((_BR, 1), lambda i: (i, 0)),
        out_shape=jax.ShapeDtypeStruct((_N, 1), jnp.float32),
        compiler_params=pltpu.CompilerParams(
            dimension_semantics=("arbitrary",),
        ),
    )(txn_x, gc, ga, ge, pc, pa, pe, W1, b1, Wv1p, bv1, Wv2, bv2)


def kernel(txn_x, idx_card, idx_addr, idx_email, mem_card, mem_addr, mem_email,
           W1, b1, unk_card, unk_addr, unk_email, Wv1, bv1, Wv2, bv2):
    pad = _NPAD - _N
    idx4 = []
    pmod = []
    for i in (idx_card, idx_addr, idx_email):
        i = i.astype(jnp.int32)
        idx4.append(jnp.pad(i // 2, (0, pad)).reshape(_NPAD // _SUB, _SUB))
        pmod.append((i % 2).reshape(_N // _BR, _BR, 1))
    m4 = [jnp.reshape(m, (m.shape[0] // 2, 128))
          for m in (mem_card, mem_addr, mem_email)]
    wv1p = Wv1.astype(jnp.bfloat16)
    gc, ga, ge = _gather3(idx4[0], idx4[1], idx4[2], m4[0], m4[1], m4[2])
    out = _mlp(txn_x, gc, ga, ge, pmod[0], pmod[1], pmod[2],
               W1.astype(jnp.bfloat16), b1.reshape(1, _H),
               wv1p, bv1.reshape(1, _H),
               Wv2, bv2.reshape(1, 1))
    return out[:, 0]
